# baseline (speedup 1.0000x reference)
import functools

import jax
import jax.numpy as jnp
from jax.experimental import pallas as pl
from jax.experimental.pallas import tpu as pltpu

B = 1
S = 2048
HIDDEN = 1024
NUM_EXPERTS = 8
TOP_K = 2
INTER = 2048

BT = 512
NT = S // BT
NE = NUM_EXPERTS + 1


def _moe_dense_body(x_ref, rw_ref, gate_ref, up_ref, down_ref, out_ref, w_ref):
    e = pl.program_id(0)
    t = pl.program_id(1)

    x = x_ref[...]

    @pl.when(e == 0)
    def _router():
        logits = jax.lax.dot_general(
            x, rw_ref[...], (((1,), (1,)), ((), ())),
            preferred_element_type=jnp.float32)
        lane = jax.lax.broadcasted_iota(jnp.int32, (BT, NUM_EXPERTS), 1)
        m1 = jnp.max(logits, axis=1, keepdims=True)
        is1 = logits == m1
        a1 = jnp.min(jnp.where(is1, lane, NUM_EXPERTS), axis=1, keepdims=True)
        sel1 = lane == a1
        neg = jnp.float32(-jnp.inf)
        l2 = jnp.where(sel1, neg, logits)
        m2 = jnp.max(l2, axis=1, keepdims=True)
        is2 = l2 == m2
        a2 = jnp.min(jnp.where(is2, lane, NUM_EXPERTS), axis=1, keepdims=True)
        sel2 = lane == a2
        e1 = jnp.exp(m1 - m1)
        e2 = jnp.exp(m2 - m1)
        denom = e1 + e2
        w8 = jnp.where(sel1, e1 / denom, 0.0) + jnp.where(sel2, e2 / denom, 0.0)
        w_ref[pl.ds(t * BT, BT), :] = w8

    xb = x.astype(jnp.bfloat16)
    g = jax.lax.dot_general(
        xb, gate_ref[0], (((1,), (1,)), ((), ())),
        preferred_element_type=jnp.float32)
    g = g * jax.lax.logistic(g)
    u = jax.lax.dot_general(
        xb, up_ref[0], (((1,), (1,)), ((), ())),
        preferred_element_type=jnp.float32)
    h = (g * u).astype(jnp.bfloat16)
    y = jax.lax.dot_general(
        h, down_ref[0], (((1,), (1,)), ((), ())),
        preferred_element_type=jnp.float32)

    w8 = w_ref[pl.ds(t * BT, BT), :]
    lane = jax.lax.broadcasted_iota(jnp.int32, (BT, NUM_EXPERTS), 1)
    we = jnp.sum(jnp.where(lane == e, w8, 0.0), axis=1, keepdims=True)
    we = jnp.where(e == NUM_EXPERTS, 1.0, we)
    yw = y * we

    @pl.when(e == 0)
    def _init():
        out_ref[pl.ds(t * BT, BT), :] = yw

    @pl.when(e > 0)
    def _acc():
        out_ref[pl.ds(t * BT, BT), :] += yw


def _moe_dense(x, router_w, gate_all, up_all, down_all, interpret=False):
    out = pl.pallas_call(
        _moe_dense_body,
        grid=(NE, NT),
        in_specs=[
            pl.BlockSpec((BT, HIDDEN), lambda e, t: (t, 0)),
            pl.BlockSpec((NUM_EXPERTS, HIDDEN), lambda e, t: (0, 0)),
            pl.BlockSpec((1, INTER, HIDDEN), lambda e, t: (e, 0, 0)),
            pl.BlockSpec((1, INTER, HIDDEN), lambda e, t: (e, 0, 0)),
            pl.BlockSpec((1, HIDDEN, INTER), lambda e, t: (e, 0, 0)),
        ],
        out_specs=pl.BlockSpec((S, HIDDEN), lambda e, t: (0, 0)),
        out_shape=jax.ShapeDtypeStruct((S, HIDDEN), jnp.float32),
        scratch_shapes=[pltpu.VMEM((S, NUM_EXPERTS), jnp.float32)],
        interpret=interpret,
    )(x, router_w, gate_all, up_all, down_all)
    return out


def kernel(hidden_states, router_w, gate_w, up_w, down_w,
           shared_gate_w, shared_up_w, shared_down_w):
    x = hidden_states.reshape(S, HIDDEN)
    gate_all = jnp.concatenate(
        [gate_w, shared_gate_w[None]], axis=0).astype(jnp.bfloat16)
    up_all = jnp.concatenate(
        [up_w, shared_up_w[None]], axis=0).astype(jnp.bfloat16)
    down_all = jnp.concatenate(
        [down_w, shared_down_w[None]], axis=0).astype(jnp.bfloat16)
    out = _moe_dense(x, router_w, gate_all, up_all, down_all)
    return (out.reshape(B, S, HIDDEN), 0.0)
